# Initial kernel scaffold; baseline (speedup 1.0000x reference)
import jax
import jax.numpy as jnp
from jax.experimental import pallas as pl


def kernel(x, expert_ids, x_active_mask, expert_scales, ori_x):
    raise NotImplementedError("write your pallas kernel here")


# speedup vs baseline: 7.9061x; 7.9061x over previous
import jax
import jax.numpy as jnp
from jax.experimental import pallas as pl


def _combine_body(scales_ref, mask_ref, x_ref, ori_ref, out_ref):
    s = jnp.sum(scales_ref[...], axis=1, keepdims=True)
    m = mask_ref[...] > 0
    out_ref[...] = jnp.where(m, s * x_ref[...], ori_ref[...])


def kernel(x, expert_ids, x_active_mask, expert_scales, ori_x):
    del expert_ids
    B, H = x.shape
    mask = x_active_mask.astype(jnp.float32).reshape(B, 1)
    return pl.pallas_call(
        _combine_body,
        out_shape=jax.ShapeDtypeStruct((B, H), x.dtype),
        grid=(8,),
        in_specs=[
            pl.BlockSpec((B, expert_scales.shape[1]), lambda i: (0, 0)),
            pl.BlockSpec((B, 1), lambda i: (0, 0)),
            pl.BlockSpec((B, H // 8), lambda i: (0, i)),
            pl.BlockSpec((B, H // 8), lambda i: (0, i)),
        ],
        out_specs=pl.BlockSpec((B, H // 8), lambda i: (0, i)),
    )(expert_scales, mask, x, ori_x)
